# baseline (speedup 1.0000x reference)
import functools

import jax
import jax.numpy as jnp
from jax import lax
from jax.experimental import pallas as pl
from jax.experimental.pallas import tpu as pltpu
from jax.experimental.pallas import tpu_sc as plsc

D = 128
V = 257
L = 1024
NC = 2
W = 512
RG = 64
SPAN = RG + W
NCHUNK = 5


def _sc_body(table_hbm, out_hbm, span_v, idx_v, sem):
    c = lax.axis_index("c")
    s = lax.axis_index("s")
    g = s // 8
    h = s % 8
    start = 960 - 512 * c + 512 * g - 64 * h

    lanes = lax.iota(jnp.int32, 16)

    def fill_idx(j, carry):
        for k in range(8):
            base = start - 895 + j * 128 + k * 16
            vec = jnp.clip(base + lanes, 0, V - 1)
            idx_v[j, pl.ds(k * 16, 16)] = vec
        return carry

    lax.fori_loop(0, NCHUNK, fill_idx, 0)
    for j in range(NCHUNK):
        pltpu.async_copy(table_hbm.at[idx_v.at[j]],
                         span_v.at[pl.ds(j * 128, 128)], sem).wait()

    row0 = 512 * c + 64 * h
    col0 = W * g

    def copy_row(q, carry):
        pltpu.sync_copy(span_v.at[pl.ds(63 - q, W)],
                        out_hbm.at[row0 + q, pl.ds(col0, W)])
        return carry

    lax.fori_loop(0, RG, copy_row, 0)


@functools.partial(
    pl.kernel,
    out_type=jax.ShapeDtypeStruct((L, L, D), jnp.float32),
    mesh=plsc.VectorSubcoreMesh(core_axis_name="c", subcore_axis_name="s"),
    scratch_types=[
        pltpu.VMEM((NCHUNK * 128, D), jnp.float32),
        pltpu.VMEM((NCHUNK, 128), jnp.int32),
        pltpu.SemaphoreType.DMA,
    ],
)
def _rel_pos_sc(table_hbm, out_hbm, span_v, idx_v, sem):
    _sc_body(table_hbm, out_hbm, span_v, idx_v, sem)


def kernel(embeddings_table, length):
    return _rel_pos_sc(embeddings_table)


# speedup vs baseline: 1.0014x; 1.0014x over previous
import functools

import jax
import jax.numpy as jnp
from jax import lax
from jax.experimental import pallas as pl
from jax.experimental.pallas import tpu as pltpu
from jax.experimental.pallas import tpu_sc as plsc

D = 128
V = 257
L = 1024
NC = 2
W = 512
RG = 64
SPAN = RG + W
NCHUNK = 5
NBUF = 8


def _sc_body(table_hbm, out_hbm, span_v, idx_v, sem):
    c = lax.axis_index("c")
    s = lax.axis_index("s")
    g = s // 8
    h = s % 8
    start = 960 - 512 * c + 512 * g - 64 * h

    lanes = lax.iota(jnp.int32, 16)

    def fill_idx(j, carry):
        for k in range(8):
            base = start - 895 + j * 128 + k * 16
            vec = jnp.clip(base + lanes, 0, V - 1)
            idx_v[j, pl.ds(k * 16, 16)] = vec
        return carry

    lax.fori_loop(0, NCHUNK, fill_idx, 0)
    for j in range(NCHUNK):
        pltpu.async_copy(table_hbm.at[idx_v.at[j]],
                         span_v.at[pl.ds(j * 128, 128)], sem).wait()

    row0 = 512 * c + 64 * h
    col0 = W * g

    inflight = []
    for q in range(RG):
        if len(inflight) >= NBUF:
            inflight.pop(0).wait()
        inflight.append(
            pltpu.async_copy(span_v.at[pl.ds(63 - q, W)],
                             out_hbm.at[row0 + q, pl.ds(col0, W)], sem))
    for cp in inflight:
        cp.wait()


@functools.partial(
    pl.kernel,
    out_type=jax.ShapeDtypeStruct((L, L, D), jnp.float32),
    mesh=plsc.VectorSubcoreMesh(core_axis_name="c", subcore_axis_name="s"),
    scratch_types=[
        pltpu.VMEM((NCHUNK * 128, D), jnp.float32),
        pltpu.VMEM((NCHUNK, 128), jnp.int32),
        pltpu.SemaphoreType.DMA,
    ],
)
def _rel_pos_sc(table_hbm, out_hbm, span_v, idx_v, sem):
    _sc_body(table_hbm, out_hbm, span_v, idx_v, sem)


def kernel(embeddings_table, length):
    return _rel_pos_sc(embeddings_table)


# speedup vs baseline: 1.2684x; 1.2666x over previous
import functools

import jax
import jax.numpy as jnp
from jax import lax
from jax.experimental import pallas as pl
from jax.experimental.pallas import tpu as pltpu
from jax.experimental.pallas import tpu_sc as plsc

D = 128
V = 257
L = 1024
P_ROWS = 2 * L - 1
FILL = L - 129
NC = 2
NS = 16
RPT = L // (NC * NS)
WS = 288
WD = L - WS
SPAN = RPT + WS
NCHUNK = (SPAN + 127) // 128
FB = 128
NBUF = 4


def _sc_body(table_hbm, out_hbm, p_sh, span_v, fill_v, trow_v, idx_v,
             sem_g, sem_s, sem_d):
    c = lax.axis_index("c")
    s = lax.axis_index("s")

    span0 = (L - 1) + WD - (512 * c + RPT * s + RPT - 1)
    lanes = lax.iota(jnp.int32, 16)

    def fill_idx(j, carry):
        for k in range(8):
            base = span0 - FILL + j * 128 + k * 16
            idx_v[j, pl.ds(k * 16, 16)] = jnp.clip(base + lanes, 0, V - 1)
        return carry

    lax.fori_loop(0, NCHUNK, fill_idx, 0)
    gathers = [
        pltpu.async_copy(table_hbm.at[idx_v.at[j]],
                         span_v.at[pl.ds(j * 128, 128)], sem_g)
        for j in range(NCHUNK)
    ]

    @pl.when(s == 0)
    def _():
        pltpu.sync_copy(table_hbm, p_sh.at[pl.ds(FILL, V)])

    def _build_fill(edge_row):
        pltpu.sync_copy(table_hbm.at[pl.ds(edge_row, 1)], trow_v)

        def rep(r, carry):
            for k in range(D // 16):
                fill_v[r, pl.ds(k * 16, 16)] = trow_v[0, pl.ds(k * 16, 16)]
            return carry

        lax.fori_loop(0, FB, rep, 0)

    @pl.when(s == 1)
    def _():
        _build_fill(0)
        pltpu.sync_copy(fill_v.at[pl.ds(0, FILL % FB)],
                        p_sh.at[pl.ds(0, FILL % FB)])
        for b in range(FILL // FB):
            pltpu.sync_copy(fill_v, p_sh.at[pl.ds(FILL % FB + b * FB, FB)])

    @pl.when(s == 2)
    def _():
        _build_fill(V - 1)
        for b in range(FILL // FB):
            pltpu.sync_copy(fill_v, p_sh.at[pl.ds(FILL + V + b * FB, FB)])
        pltpu.sync_copy(fill_v.at[pl.ds(0, FILL % FB)],
                        p_sh.at[pl.ds(P_ROWS - FILL % FB, FILL % FB)])

    for cp in gathers:
        cp.wait()
    plsc.subcore_barrier()

    row0 = 512 * c + RPT * s
    dma_q, str_q = [], []
    for r in range(RPT):
        i = row0 + r
        if len(str_q) >= NBUF:
            str_q.pop(0).wait()
        str_q.append(
            pltpu.async_copy(span_v.at[pl.ds(RPT - 1 - r, WS)],
                             out_hbm.at[i, pl.ds(WD, WS)], sem_s))
        if len(dma_q) >= NBUF:
            dma_q.pop(0).wait()
        dma_q.append(
            pltpu.async_copy(p_sh.at[pl.ds(L - 1 - i, WD)],
                             out_hbm.at[i, pl.ds(0, WD)], sem_d))
    for cp in str_q + dma_q:
        cp.wait()


@functools.partial(
    pl.kernel,
    out_type=jax.ShapeDtypeStruct((L, L, D), jnp.float32),
    mesh=plsc.VectorSubcoreMesh(core_axis_name="c", subcore_axis_name="s"),
    scratch_types=[
        pltpu.VMEM_SHARED((P_ROWS, D), jnp.float32),
        pltpu.VMEM((NCHUNK * 128, D), jnp.float32),
        pltpu.VMEM((FB, D), jnp.float32),
        pltpu.VMEM((1, D), jnp.float32),
        pltpu.VMEM((NCHUNK, 128), jnp.int32),
        pltpu.SemaphoreType.DMA,
        pltpu.SemaphoreType.DMA,
        pltpu.SemaphoreType.DMA,
    ],
)
def _rel_pos_sc(table_hbm, out_hbm, p_sh, span_v, fill_v, trow_v, idx_v,
                sem_g, sem_s, sem_d):
    _sc_body(table_hbm, out_hbm, p_sh, span_v, fill_v, trow_v, idx_v,
             sem_g, sem_s, sem_d)


def kernel(embeddings_table, length):
    return _rel_pos_sc(embeddings_table)


# speedup vs baseline: 2.6387x; 2.0803x over previous
import functools

import jax
import jax.numpy as jnp
from jax import lax
from jax.experimental import pallas as pl
from jax.experimental.pallas import tpu as pltpu
from jax.experimental.pallas import tpu_sc as plsc

D = 128
V = 257
L = 1024
P_ROWS = 2 * L - 1
FILL = L - 129
NC = 2
NS = 16
ROWS_PER_TILE = L // (NC * NS)
FB = 128
NBUF = 2


def _sc_body(table_hbm, out_hbm, p_sh, fill_v, trow_v, sem_o):
    c = lax.axis_index("c")
    s = lax.axis_index("s")

    @pl.when(s == 0)
    def _():
        pltpu.sync_copy(table_hbm, p_sh.at[pl.ds(FILL, V)])

    def _build_fill(edge_row):
        pltpu.sync_copy(table_hbm.at[pl.ds(edge_row, 1)], trow_v)

        def rep(r, carry):
            for k in range(D // 16):
                fill_v[r, pl.ds(k * 16, 16)] = trow_v[0, pl.ds(k * 16, 16)]
            return carry

        lax.fori_loop(0, FB, rep, 0)

    @pl.when(s == 1)
    def _():
        _build_fill(0)
        pltpu.sync_copy(fill_v.at[pl.ds(0, FILL % FB)],
                        p_sh.at[pl.ds(0, FILL % FB)])
        for b in range(FILL // FB):
            pltpu.sync_copy(fill_v, p_sh.at[pl.ds(FILL % FB + b * FB, FB)])

    @pl.when(s == 2)
    def _():
        _build_fill(V - 1)
        for b in range(FILL // FB):
            pltpu.sync_copy(fill_v, p_sh.at[pl.ds(FILL + V + b * FB, FB)])
        pltpu.sync_copy(fill_v.at[pl.ds(0, FILL % FB)],
                        p_sh.at[pl.ds(P_ROWS - FILL % FB, FILL % FB)])

    plsc.subcore_barrier()

    base = c * (NS * ROWS_PER_TILE) + s * ROWS_PER_TILE

    inflight = []
    for k in range(ROWS_PER_TILE):
        i = base + k
        if len(inflight) >= NBUF:
            inflight.pop(0).wait()
        inflight.append(
            pltpu.async_copy(p_sh.at[pl.ds(L - 1 - i, L)],
                             out_hbm.at[i], sem_o))
    for cp in inflight:
        cp.wait()


@functools.partial(
    pl.kernel,
    out_type=jax.ShapeDtypeStruct((L, L, D), jnp.float32),
    mesh=plsc.VectorSubcoreMesh(core_axis_name="c", subcore_axis_name="s"),
    scratch_types=[
        pltpu.VMEM_SHARED((P_ROWS, D), jnp.float32),
        pltpu.VMEM((FB, D), jnp.float32),
        pltpu.VMEM((1, D), jnp.float32),
        pltpu.SemaphoreType.DMA,
    ],
)
def _rel_pos_sc(table_hbm, out_hbm, p_sh, fill_v, trow_v, sem_o):
    _sc_body(table_hbm, out_hbm, p_sh, fill_v, trow_v, sem_o)


def kernel(embeddings_table, length):
    return _rel_pos_sc(embeddings_table)
